# baseline (speedup 1.0000x reference)
import functools

import jax
import jax.numpy as jnp
from jax import lax
from jax.experimental import pallas as pl
from jax.experimental.pallas import tpu as pltpu
from jax.experimental.pallas import tpu_sc as plsc

N = 100000
B = 50
L = 50
NC = B * L
IN_CH = 128
SW = 64
GW = 64

ROW_BLK = 2000


def _mm_relu_body(x_ref, w_ref, b_ref, o_ref):
    o_ref[...] = jax.nn.relu(
        jnp.dot(x_ref[...], w_ref[...], preferred_element_type=jnp.float32)
        + b_ref[...]
    )


def _mm_relu(x, w, b):
    n, d = x.shape
    _, dout = w.shape
    grid = (n // ROW_BLK,)
    return pl.pallas_call(
        _mm_relu_body,
        grid=grid,
        in_specs=[
            pl.BlockSpec((ROW_BLK, d), lambda i: (i, 0)),
            pl.BlockSpec((d, dout), lambda i: (0, 0)),
            pl.BlockSpec((dout,), lambda i: (0,)),
        ],
        out_specs=pl.BlockSpec((ROW_BLK, dout), lambda i: (i, 0)),
        out_shape=jax.ShapeDtypeStruct((n, dout), jnp.float32),
    )(x, w, b)


def _mm2_relu_body(h_ref, g_ref, wa_ref, wb_ref, b_ref, o_ref):
    acc = jnp.dot(h_ref[...], wa_ref[...], preferred_element_type=jnp.float32)
    acc += jnp.dot(g_ref[...], wb_ref[...], preferred_element_type=jnp.float32)
    o_ref[...] = jax.nn.relu(acc + b_ref[...])


def _mm2_relu(h, g, wa, wb, b):
    n, d = h.shape
    _, dout = wa.shape
    grid = (n // ROW_BLK,)
    return pl.pallas_call(
        _mm2_relu_body,
        grid=grid,
        in_specs=[
            pl.BlockSpec((ROW_BLK, d), lambda i: (i, 0)),
            pl.BlockSpec((ROW_BLK, d), lambda i: (i, 0)),
            pl.BlockSpec((d, dout), lambda i: (0, 0)),
            pl.BlockSpec((d, dout), lambda i: (0, 0)),
            pl.BlockSpec((dout,), lambda i: (0,)),
        ],
        out_specs=pl.BlockSpec((ROW_BLK, dout), lambda i: (i, 0)),
        out_shape=jax.ShapeDtypeStruct((n, dout), jnp.float32),
    )(h, g, wa, wb, b)


_NP = 102400
_RPW = 3200
_GCH = 128
_NCP = 2560


def _sc_gather(table, idx):
    mesh = plsc.VectorSubcoreMesh(core_axis_name="c", subcore_axis_name="s")

    @functools.partial(
        pl.kernel,
        out_type=jax.ShapeDtypeStruct((_NP, 128), jnp.float32),
        mesh=mesh,
        scratch_types=[
            pltpu.VMEM((_GCH,), jnp.int32),
            pltpu.VMEM((_GCH, 128), jnp.float32),
            pltpu.SemaphoreType.DMA,
        ],
    )
    def k(table_hbm, idx_hbm, out_hbm, idx_v, rows_v, sem):
        wid = lax.axis_index("s") * 2 + lax.axis_index("c")
        base = wid * _RPW

        def body(j, carry):
            off = base + j * _GCH
            pltpu.sync_copy(idx_hbm.at[pl.ds(off, _GCH)], idx_v)
            pltpu.async_copy(table_hbm.at[idx_v], rows_v, sem).wait()
            pltpu.sync_copy(rows_v, out_hbm.at[pl.ds(off, _GCH)])
            return carry

        lax.fori_loop(0, _RPW // _GCH, body, 0)

    return k(table, idx)


def _seg_max(h, seg, num):
    m = jax.ops.segment_max(h, seg, num_segments=num)
    return jnp.where(jnp.isneginf(m), 0.0, m)


def kernel(x, identifier, cluster, valid_len, W0, b0, W1, b1, W2, b2, Wq, bq, Wk, bk, Wv, bv):
    cluster_pad = jnp.pad(cluster, (0, _NP - N), constant_values=NC)

    def _gather(M):
        Mp = jnp.pad(M, ((0, _NCP - NC), (0, SW)))
        return _sc_gather(Mp, cluster_pad)[:N, :SW]

    h0 = _mm_relu(x, W0, b0)
    M0 = _seg_max(h0, cluster, NC)
    h1 = _mm2_relu(h0, _gather(M0), W1[:SW], W1[SW:], b1)
    M1 = _seg_max(h1, cluster, NC)
    h2 = _mm2_relu(h1, _gather(M1), W2[:SW], W2[SW:], b2)
    M2 = _seg_max(h2, cluster, NC)
    sub = jnp.concatenate([M2, M2], axis=1)
    sub = sub / jnp.maximum(jnp.linalg.norm(sub, axis=1, keepdims=True), 1e-12)
    xg = jnp.concatenate([sub, identifier], axis=1).reshape(B, L, 2 * SW + 2)
    q = xg @ Wq + bq
    k = xg @ Wk + bk
    v = xg @ Wv + bv
    scores = jnp.einsum('bqd,bkd->bqk', q, k) / jnp.sqrt(jnp.float32(GW))
    mask = jnp.arange(L)[None, :] < valid_len[:, None]
    scores = jnp.where(mask[:, None, :], scores, -1e9)
    attn = jax.nn.softmax(scores, axis=-1)
    return jnp.einsum('bqk,bkd->bqd', attn, v)


# speedup vs baseline: 2.3573x; 2.3573x over previous
import functools

import jax
import jax.numpy as jnp
from jax import lax
from jax.experimental import pallas as pl
from jax.experimental.pallas import tpu as pltpu
from jax.experimental.pallas import tpu_sc as plsc

N = 100000
B = 50
L = 50
NC = B * L
SW = 64
GW = 64

_NW = 32
_NP = 102400
_RPW = 3200
_GCH = 128
_CW = 80
_NCP = 2560
_SLEN = 2576
_CH = 256
_BLK = 2048
_GRID = 49

_mesh = plsc.VectorSubcoreMesh(core_axis_name="c", subcore_axis_name="s")
_NEG = -1.0e30


def _wid():
    return lax.axis_index("s") * 2 + lax.axis_index("c")


def _iota16():
    return lax.broadcasted_iota(jnp.int32, (16,), 0)


def _sget(ref, i):
    return ref[pl.ds(i, 16)][0]


def _sc_prep(cluster_pad):
    @functools.partial(
        pl.kernel,
        out_type=jax.ShapeDtypeStruct((_SLEN,), jnp.int32),
        mesh=_mesh,
        scratch_types=[
            pltpu.VMEM((96,), jnp.int32),
            pltpu.VMEM((96,), jnp.int32),
            pltpu.VMEM((96,), jnp.int32),
            pltpu.SemaphoreType.DMA,
        ],
    )
    def k(cl_hbm, s_hbm, idx_v, val_v, res_v, sem):
        w = _wid()
        cbase = pl.multiple_of(w * _CW, 16)
        it16 = _iota16()
        cs = [cbase + 16 * j + it16 for j in range(6)]
        pos = [jnp.zeros((16,), jnp.int32) for _ in range(6)]
        for st in range(16, -1, -1):
            step = jnp.int32(1 << st)
            cands = []
            for j in range(6):
                cand = jnp.minimum(pos[j] + step, jnp.int32(_NP))
                cands.append(cand)
                idx_v[pl.ds(16 * j, 16)] = cand - 1
            pltpu.async_copy(cl_hbm.at[idx_v], val_v, sem).wait()
            for j in range(6):
                pred = val_v[pl.ds(16 * j, 16)] < cs[j]
                pos[j] = jnp.where(pred, cands[j], pos[j])
        for j in range(6):
            res_v[pl.ds(16 * j, 16)] = pos[j]

        @pl.when(w == _NW - 1)
        def _():
            pltpu.sync_copy(res_v, s_hbm.at[pl.ds(cbase, 96)])

        @pl.when(w != _NW - 1)
        def _():
            pltpu.sync_copy(res_v.at[pl.ds(0, 80)], s_hbm.at[pl.ds(cbase, 80)])

    return k(cluster_pad)


def _sc_segmax(u, S, init):
    @functools.partial(
        pl.kernel,
        out_type=jax.ShapeDtypeStruct((_NCP, 128), jnp.float32),
        mesh=_mesh,
        scratch_types=[
            pltpu.VMEM((128,), jnp.int32),
            pltpu.VMEM((_CH, 128), jnp.float32),
            pltpu.VMEM((_CW, 128), jnp.float32),
            pltpu.SemaphoreType.DMA,
        ],
    )
    def k(u_hbm, s_hbm, t_hbm, sv, ubuf, tbuf, sem):
        w = _wid()
        c0 = pl.multiple_of(w * _CW, 16)
        pltpu.sync_copy(s_hbm.at[pl.ds(c0, 96)], sv.at[pl.ds(0, 96)])
        initv = jnp.full((16,), init, jnp.float32)
        zv = jnp.zeros((16,), jnp.float32)

        def prefill(i, carry):
            for t in range(4):
                tbuf[i, pl.ds(16 * t, 16)] = initv
            for t in range(4, 8):
                tbuf[i, pl.ds(16 * t, 16)] = zv
            return carry

        lax.fori_loop(0, _CW, prefill, 0)

        rs = jnp.bitwise_and(_sget(sv, jnp.int32(0)), jnp.int32(-8))
        re = _sget(sv, jnp.int32(_CW))
        nch = lax.div(re - rs + jnp.int32(_CH - 1), jnp.int32(_CH))

        def chunk(kk, cl):
            base = jnp.minimum(rs + kk * _CH, jnp.int32(_NP - _CH))
            base = pl.multiple_of(base, 8)
            pltpu.sync_copy(u_hbm.at[pl.ds(base, _CH)], ubuf)
            lim = base + _CH

            def redmem(cidx, s, e):
                def rbody(r, z):
                    off = r - base
                    for t in range(4):
                        tbuf[cidx, pl.ds(16 * t, 16)] = jnp.maximum(
                            tbuf[cidx, pl.ds(16 * t, 16)],
                            ubuf[off, pl.ds(16 * t, 16)],
                        )
                    return z

                lax.fori_loop(s, e, rbody, 0)

            def bit(_, lh):
                lo, hi = lh
                mid = lax.div(lo + hi, jnp.int32(2))
                gt = _sget(sv, mid + 1) > lim
                lo2 = jnp.where(gt, lo, mid + 1)
                hi2 = jnp.where(gt, mid, hi)
                keep = lo < hi
                return (jnp.where(keep, lo2, lo), jnp.where(keep, hi2, hi))

            cl_new, _ = lax.fori_loop(0, 7, bit, (cl, jnp.int32(_CW)))

            def cbody(c, z):
                s = jnp.maximum(_sget(sv, c), base)
                redmem(c, s, _sget(sv, c + 1))
                return z

            lax.fori_loop(cl, cl_new, cbody, 0)
            cl = cl_new
            cli = jnp.minimum(cl, jnp.int32(_CW - 1))
            s = jnp.maximum(_sget(sv, cli), base)
            e = jnp.minimum(_sget(sv, cli + 1), lim)
            e = jnp.where(cl < _CW, jnp.maximum(e, s), s)
            redmem(cli, s, e)
            return cl

        lax.fori_loop(0, nch, chunk, jnp.int32(0))
        pltpu.sync_copy(tbuf, t_hbm.at[pl.ds(c0, _CW)])

    return k(u, S)


def _sc_gather(table, idx):
    @functools.partial(
        pl.kernel,
        out_type=jax.ShapeDtypeStruct((_NP, 128), jnp.float32),
        mesh=_mesh,
        scratch_types=[
            pltpu.VMEM((_GCH,), jnp.int32),
            pltpu.VMEM((_GCH, 128), jnp.float32),
            pltpu.SemaphoreType.DMA,
        ],
    )
    def k(table_hbm, idx_hbm, out_hbm, idx_v, rows_v, sem):
        base = _wid() * _RPW

        def body(j, carry):
            off = base + j * _GCH
            pltpu.sync_copy(idx_hbm.at[pl.ds(off, _GCH)], idx_v)
            pltpu.async_copy(table_hbm.at[idx_v], rows_v, sem).wait()
            pltpu.sync_copy(rows_v, out_hbm.at[pl.ds(off, _GCH)])
            return carry

        lax.fori_loop(0, _RPW // _GCH, body, 0)

    return k(table, idx)


def _mm0_body(x_ref, w_ref, b_ref, o_ref):
    acc = jnp.dot(x_ref[...], w_ref[...], preferred_element_type=jnp.float32)
    h = jax.nn.relu(acc + b_ref[...])
    o_ref[...] = jnp.concatenate(
        [h, jnp.zeros((h.shape[0], 64), jnp.float32)], axis=1
    )


def _tc_mm0(x, w, b):
    return pl.pallas_call(
        _mm0_body,
        grid=(_GRID,),
        in_specs=[
            pl.BlockSpec((_BLK, 128), lambda i: (i, 0)),
            pl.BlockSpec((128, 64), lambda i: (0, 0)),
            pl.BlockSpec((64,), lambda i: (0,)),
        ],
        out_specs=pl.BlockSpec((_BLK, 128), lambda i: (i, 0)),
        out_shape=jax.ShapeDtypeStruct((_NP, 128), jnp.float32),
    )(x, w, b)


def _mm1_body(h_ref, w_ref, b_ref, o_ref):
    acc = jnp.dot(h_ref[...], w_ref[...], preferred_element_type=jnp.float32)
    u = acc + b_ref[...]
    o_ref[...] = jnp.concatenate(
        [u, jnp.zeros((u.shape[0], 64), jnp.float32)], axis=1
    )


def _tc_mm1(h, w, b):
    return pl.pallas_call(
        _mm1_body,
        grid=(_GRID,),
        in_specs=[
            pl.BlockSpec((_BLK, 128), lambda i: (i, 0)),
            pl.BlockSpec((128, 64), lambda i: (0, 0)),
            pl.BlockSpec((64,), lambda i: (0,)),
        ],
        out_specs=pl.BlockSpec((_BLK, 128), lambda i: (i, 0)),
        out_shape=jax.ShapeDtypeStruct((_NP, 128), jnp.float32),
    )(h, w, b)


def _mm2_body(u_ref, e_ref, w_ref, b_ref, o_ref):
    h = jax.nn.relu(u_ref[...] + e_ref[...])
    acc = jnp.dot(h, w_ref[...], preferred_element_type=jnp.float32)
    u = acc + b_ref[...]
    o_ref[...] = jnp.concatenate(
        [u, jnp.zeros((u.shape[0], 64), jnp.float32)], axis=1
    )


def _tc_mm2(u1, e1, w, b):
    return pl.pallas_call(
        _mm2_body,
        grid=(_GRID,),
        in_specs=[
            pl.BlockSpec((_BLK, 128), lambda i: (i, 0)),
            pl.BlockSpec((_BLK, 128), lambda i: (i, 0)),
            pl.BlockSpec((128, 64), lambda i: (0, 0)),
            pl.BlockSpec((64,), lambda i: (0,)),
        ],
        out_specs=pl.BlockSpec((_BLK, 128), lambda i: (i, 0)),
        out_shape=jax.ShapeDtypeStruct((_NP, 128), jnp.float32),
    )(u1, e1, w, b)


def _tinymm_body(t_ref, a_ref, w_ref, o_ref):
    m = jax.nn.relu(t_ref[...] + a_ref[...])
    acc = jnp.dot(m, w_ref[...], preferred_element_type=jnp.float32)
    o_ref[...] = jnp.concatenate(
        [acc, jnp.zeros((acc.shape[0], 64), jnp.float32)], axis=1
    )


def _tc_tinymm(t, a, w):
    return pl.pallas_call(
        _tinymm_body,
        in_specs=[
            pl.BlockSpec((_NCP, 128), lambda: (0, 0)),
            pl.BlockSpec((_NCP, 128), lambda: (0, 0)),
            pl.BlockSpec((128, 64), lambda: (0, 0)),
        ],
        out_specs=pl.BlockSpec((_NCP, 128), lambda: (0, 0)),
        out_shape=jax.ShapeDtypeStruct((_NCP, 128), jnp.float32),
    )(t, a, w)


def _qkv_body(t_ref, q_ref, id_ref, wq_ref, wk_ref, wv_ref, w2_ref, bqkv_ref,
              oq_ref, ok_ref, ov_ref):
    m = jax.nn.relu(t_ref[...] + q_ref[...])
    n2 = 2.0 * jnp.sum(m * m, axis=1, keepdims=True)
    inv = 1.0 / jnp.maximum(jnp.sqrt(n2), 1e-12)
    idc = id_ref[...]
    for wref, oref, col in ((wq_ref, oq_ref, 0), (wk_ref, ok_ref, 1),
                            (wv_ref, ov_ref, 2)):
        z = jnp.dot(m, wref[...], preferred_element_type=jnp.float32)
        zid = jnp.dot(idc, w2_ref[pl.ds(2 * col, 2), :],
                      preferred_element_type=jnp.float32)
        oref[...] = z * inv + zid + bqkv_ref[col, :]


def _tc_qkv(t2, q1, idp, wqs, wks, wvs, w2s, bqkv):
    return pl.pallas_call(
        _qkv_body,
        in_specs=[
            pl.BlockSpec((_NCP, 128), lambda: (0, 0)),
            pl.BlockSpec((_NCP, 128), lambda: (0, 0)),
            pl.BlockSpec((_NCP, 2), lambda: (0, 0)),
            pl.BlockSpec((128, 64), lambda: (0, 0)),
            pl.BlockSpec((128, 64), lambda: (0, 0)),
            pl.BlockSpec((128, 64), lambda: (0, 0)),
            pl.BlockSpec((6, 64), lambda: (0, 0)),
            pl.BlockSpec((3, 64), lambda: (0, 0)),
        ],
        out_specs=[
            pl.BlockSpec((_NCP, 64), lambda: (0, 0)),
            pl.BlockSpec((_NCP, 64), lambda: (0, 0)),
            pl.BlockSpec((_NCP, 64), lambda: (0, 0)),
        ],
        out_shape=[jax.ShapeDtypeStruct((_NCP, 64), jnp.float32)] * 3,
    )(t2, q1, idp, wqs, wks, wvs, w2s, bqkv)


def _attn_body(q_ref, k_ref, v_ref, vl_ref, o_ref):
    b = pl.program_id(0)
    qb = q_ref[0]
    kb = k_ref[0]
    vb = v_ref[0]
    scores = lax.dot_general(
        qb, kb, (((1,), (1,)), ((), ())), preferred_element_type=jnp.float32
    ) * 0.125
    valid = lax.broadcasted_iota(jnp.int32, (L, L), 1) < vl_ref[b]
    scores = jnp.where(valid, scores, -1e9)
    mx = jnp.max(scores, axis=-1, keepdims=True)
    e = jnp.exp(scores - mx)
    p = e / jnp.sum(e, axis=-1, keepdims=True)
    o_ref[0] = jnp.dot(p, vb, preferred_element_type=jnp.float32)


def _tc_attn(q3, k3, v3, valid_len):
    return pl.pallas_call(
        _attn_body,
        grid=(B,),
        in_specs=[
            pl.BlockSpec((1, L, GW), lambda b: (b, 0, 0)),
            pl.BlockSpec((1, L, GW), lambda b: (b, 0, 0)),
            pl.BlockSpec((1, L, GW), lambda b: (b, 0, 0)),
            pl.BlockSpec(memory_space=pltpu.SMEM),
        ],
        out_specs=pl.BlockSpec((1, L, GW), lambda b: (b, 0, 0)),
        out_shape=jax.ShapeDtypeStruct((B, L, GW), jnp.float32),
    )(q3, k3, v3, valid_len)


def kernel(x, identifier, cluster, valid_len, W0, b0, W1, b1, W2, b2, Wq, bq, Wk, bk, Wv, bv):
    f32 = jnp.float32
    cluster_pad = jnp.pad(cluster, (0, _NP - N), constant_values=NC)
    z64 = jnp.zeros((64, 64), f32)
    W1a = jnp.concatenate([W1[:SW], z64], axis=0)
    W1b = jnp.concatenate([W1[SW:], z64], axis=0)
    W2a = jnp.concatenate([W2[:SW], z64], axis=0)
    W2b = jnp.concatenate([W2[SW:], z64], axis=0)
    wqs = jnp.concatenate([Wq[:SW] + Wq[SW:2 * SW], z64], axis=0)
    wks = jnp.concatenate([Wk[:SW] + Wk[SW:2 * SW], z64], axis=0)
    wvs = jnp.concatenate([Wv[:SW] + Wv[SW:2 * SW], z64], axis=0)
    w2s = jnp.concatenate([Wq[2 * SW:], Wk[2 * SW:], Wv[2 * SW:]], axis=0)
    bqkv = jnp.stack([bq, bk, bv], axis=0)
    idp = jnp.pad(identifier, ((0, _NCP - NC), (0, 0)))
    zero_ncp = jnp.zeros((_NCP, 128), f32)

    S = _sc_prep(cluster_pad)
    h0 = _tc_mm0(x, W0, b0)
    T0 = _sc_segmax(h0, S, 0.0)
    u1 = _tc_mm1(h0, W1a, b1)
    Q0 = _tc_tinymm(T0, zero_ncp, W1b)
    E1 = _sc_gather(Q0, cluster_pad)
    T1 = _sc_segmax(u1, S, _NEG)
    u2 = _tc_mm2(u1, E1, W2a, b2)
    Q1 = _tc_tinymm(T1, Q0, W2b)
    T2 = _sc_segmax(u2, S, _NEG)
    q, k, v = _tc_qkv(T2, Q1, idp, wqs, wks, wvs, w2s, bqkv)
    q3 = q[:NC].reshape(B, L, GW)
    k3 = k[:NC].reshape(B, L, GW)
    v3 = v[:NC].reshape(B, L, GW)
    return _tc_attn(q3, k3, v3, valid_len)


# speedup vs baseline: 4.2486x; 1.8023x over previous
import functools

import jax
import jax.numpy as jnp
from jax import lax
from jax.experimental import pallas as pl
from jax.experimental.pallas import tpu as pltpu
from jax.experimental.pallas import tpu_sc as plsc

N = 100000
B = 50
L = 50
NC = B * L
SW = 64
GW = 64

_NW = 32
_NP = 102400
_RPW = 3200
_GCH = 128
_CW = 80
_NCP = 2560
_SLEN = 2576
_CH = 256
_BLK = 2048
_GRID = 49

_mesh = plsc.VectorSubcoreMesh(core_axis_name="c", subcore_axis_name="s")
_NEG = -1.0e30


def _wid():
    return lax.axis_index("s") * 2 + lax.axis_index("c")


def _iota16():
    return lax.broadcasted_iota(jnp.int32, (16,), 0)


def _sget(ref, i):
    return ref[pl.ds(i, 16)][0]


def _sc_prep(cluster_pad):
    @functools.partial(
        pl.kernel,
        out_type=jax.ShapeDtypeStruct((_SLEN,), jnp.int32),
        mesh=_mesh,
        scratch_types=[
            pltpu.VMEM((96,), jnp.int32),
            pltpu.VMEM((96,), jnp.int32),
            pltpu.VMEM((96,), jnp.int32),
            pltpu.SemaphoreType.DMA,
        ],
    )
    def k(cl_hbm, s_hbm, idx_v, val_v, res_v, sem):
        w = _wid()
        cbase = pl.multiple_of(w * _CW, 16)
        it16 = _iota16()
        cs = [cbase + 16 * j + it16 for j in range(6)]
        pos = [jnp.zeros((16,), jnp.int32) for _ in range(6)]
        for st in range(16, -1, -1):
            step = jnp.int32(1 << st)
            cands = []
            for j in range(6):
                cand = jnp.minimum(pos[j] + step, jnp.int32(_NP))
                cands.append(cand)
                idx_v[pl.ds(16 * j, 16)] = cand - 1
            pltpu.async_copy(cl_hbm.at[idx_v], val_v, sem).wait()
            for j in range(6):
                pred = val_v[pl.ds(16 * j, 16)] < cs[j]
                pos[j] = jnp.where(pred, cands[j], pos[j])
        for j in range(6):
            res_v[pl.ds(16 * j, 16)] = pos[j]

        @pl.when(w == _NW - 1)
        def _():
            pltpu.sync_copy(res_v, s_hbm.at[pl.ds(cbase, 96)])

        @pl.when(w != _NW - 1)
        def _():
            pltpu.sync_copy(res_v.at[pl.ds(0, 80)], s_hbm.at[pl.ds(cbase, 80)])

    return k(cluster_pad)


def _sc_segmax(u, S, init):
    @functools.partial(
        pl.kernel,
        out_type=jax.ShapeDtypeStruct((_NCP, 128), jnp.float32),
        mesh=_mesh,
        scratch_types=[
            pltpu.VMEM((128,), jnp.int32),
            pltpu.VMEM((_CH, 128), jnp.float32),
            pltpu.VMEM((_CW, 128), jnp.float32),
            pltpu.SemaphoreType.DMA,
        ],
    )
    def k(u_hbm, s_hbm, t_hbm, sv, ubuf, tbuf, sem):
        w = _wid()
        c0 = pl.multiple_of(w * _CW, 16)
        pltpu.sync_copy(s_hbm.at[pl.ds(c0, 96)], sv.at[pl.ds(0, 96)])
        initv = jnp.full((16,), init, jnp.float32)
        zv = jnp.zeros((16,), jnp.float32)

        def prefill(i, carry):
            for t in range(4):
                tbuf[i, pl.ds(16 * t, 16)] = initv
            for t in range(4, 8):
                tbuf[i, pl.ds(16 * t, 16)] = zv
            return carry

        lax.fori_loop(0, _CW, prefill, 0)

        rs = jnp.bitwise_and(_sget(sv, jnp.int32(0)), jnp.int32(-8))
        re = _sget(sv, jnp.int32(_CW))
        nch = lax.div(re - rs + jnp.int32(_CH - 1), jnp.int32(_CH))

        def chunk(kk, cl):
            base = jnp.minimum(rs + kk * _CH, jnp.int32(_NP - _CH))
            base = pl.multiple_of(base, 8)
            pltpu.sync_copy(u_hbm.at[pl.ds(base, _CH)], ubuf)
            lim = base + _CH

            def redmem(cidx, s, e):
                def rbody(r, acc):
                    off = r - base
                    return tuple(
                        jnp.maximum(acc[t], ubuf[off, pl.ds(16 * t, 16)])
                        for t in range(4)
                    )

                acc0 = tuple(tbuf[cidx, pl.ds(16 * t, 16)] for t in range(4))
                acc = lax.fori_loop(s, e, rbody, acc0)
                for t in range(4):
                    tbuf[cidx, pl.ds(16 * t, 16)] = acc[t]

            def bit(_, lh):
                lo, hi = lh
                mid = lax.div(lo + hi, jnp.int32(2))
                gt = _sget(sv, mid + 1) > lim
                lo2 = jnp.where(gt, lo, mid + 1)
                hi2 = jnp.where(gt, mid, hi)
                keep = lo < hi
                return (jnp.where(keep, lo2, lo), jnp.where(keep, hi2, hi))

            cl_new, _ = lax.fori_loop(0, 7, bit, (cl, jnp.int32(_CW)))

            def cbody(c, z):
                s = jnp.maximum(_sget(sv, c), base)
                redmem(c, s, _sget(sv, c + 1))
                return z

            lax.fori_loop(cl, cl_new, cbody, 0)
            cl = cl_new
            cli = jnp.minimum(cl, jnp.int32(_CW - 1))
            s = jnp.maximum(_sget(sv, cli), base)
            e = jnp.minimum(_sget(sv, cli + 1), lim)
            e = jnp.where(cl < _CW, jnp.maximum(e, s), s)
            redmem(cli, s, e)
            return cl

        lax.fori_loop(0, nch, chunk, jnp.int32(0))
        pltpu.sync_copy(tbuf, t_hbm.at[pl.ds(c0, _CW)])

    return k(u, S)


def _sc_gather(table, idx):
    @functools.partial(
        pl.kernel,
        out_type=jax.ShapeDtypeStruct((_NP, 128), jnp.float32),
        mesh=_mesh,
        scratch_types=[
            pltpu.VMEM((_GCH,), jnp.int32),
            pltpu.VMEM((_GCH, 128), jnp.float32),
            pltpu.VMEM_SHARED((_NCP, 128), jnp.float32),
            pltpu.SemaphoreType.DMA,
        ],
    )
    def k(table_hbm, idx_hbm, out_hbm, idx_v, rows_v, qtab, sem):
        base = _wid() * _RPW

        @pl.when(lax.axis_index("s") == 0)
        def _():
            pltpu.sync_copy(table_hbm, qtab)

        plsc.subcore_barrier()

        def body(j, carry):
            off = base + j * _GCH
            pltpu.sync_copy(idx_hbm.at[pl.ds(off, _GCH)], idx_v)
            pltpu.async_copy(qtab.at[idx_v], rows_v, sem).wait()
            pltpu.sync_copy(rows_v, out_hbm.at[pl.ds(off, _GCH)])
            return carry

        lax.fori_loop(0, _RPW // _GCH, body, 0)

    return k(table, idx)


def _mm0_body(x_ref, w_ref, b_ref, o_ref):
    acc = jnp.dot(x_ref[...], w_ref[...], preferred_element_type=jnp.float32)
    h = jax.nn.relu(acc + b_ref[...])
    o_ref[...] = jnp.concatenate(
        [h, jnp.zeros((h.shape[0], 64), jnp.float32)], axis=1
    )


def _tc_mm0(x, w, b):
    return pl.pallas_call(
        _mm0_body,
        grid=(_GRID,),
        in_specs=[
            pl.BlockSpec((_BLK, 128), lambda i: (i, 0)),
            pl.BlockSpec((128, 64), lambda i: (0, 0)),
            pl.BlockSpec((64,), lambda i: (0,)),
        ],
        out_specs=pl.BlockSpec((_BLK, 128), lambda i: (i, 0)),
        out_shape=jax.ShapeDtypeStruct((_NP, 128), jnp.float32),
    )(x, w, b)


def _mm1_body(h_ref, w_ref, b_ref, o_ref):
    acc = jnp.dot(h_ref[...], w_ref[...], preferred_element_type=jnp.float32)
    u = acc + b_ref[...]
    o_ref[...] = jnp.concatenate(
        [u, jnp.zeros((u.shape[0], 64), jnp.float32)], axis=1
    )


def _tc_mm1(h, w, b):
    return pl.pallas_call(
        _mm1_body,
        grid=(_GRID,),
        in_specs=[
            pl.BlockSpec((_BLK, 128), lambda i: (i, 0)),
            pl.BlockSpec((128, 64), lambda i: (0, 0)),
            pl.BlockSpec((64,), lambda i: (0,)),
        ],
        out_specs=pl.BlockSpec((_BLK, 128), lambda i: (i, 0)),
        out_shape=jax.ShapeDtypeStruct((_NP, 128), jnp.float32),
    )(h, w, b)


def _mm2_body(u_ref, e_ref, w_ref, b_ref, o_ref):
    h = jax.nn.relu(u_ref[...] + e_ref[...])
    acc = jnp.dot(h, w_ref[...], preferred_element_type=jnp.float32)
    u = acc + b_ref[...]
    o_ref[...] = jnp.concatenate(
        [u, jnp.zeros((u.shape[0], 64), jnp.float32)], axis=1
    )


def _tc_mm2(u1, e1, w, b):
    return pl.pallas_call(
        _mm2_body,
        grid=(_GRID,),
        in_specs=[
            pl.BlockSpec((_BLK, 128), lambda i: (i, 0)),
            pl.BlockSpec((_BLK, 128), lambda i: (i, 0)),
            pl.BlockSpec((128, 64), lambda i: (0, 0)),
            pl.BlockSpec((64,), lambda i: (0,)),
        ],
        out_specs=pl.BlockSpec((_BLK, 128), lambda i: (i, 0)),
        out_shape=jax.ShapeDtypeStruct((_NP, 128), jnp.float32),
    )(u1, e1, w, b)


def _tinymm_body(t_ref, a_ref, w_ref, o_ref):
    m = jax.nn.relu(t_ref[...] + a_ref[...])
    acc = jnp.dot(m, w_ref[...], preferred_element_type=jnp.float32)
    o_ref[...] = jnp.concatenate(
        [acc, jnp.zeros((acc.shape[0], 64), jnp.float32)], axis=1
    )


def _tc_tinymm(t, a, w):
    return pl.pallas_call(
        _tinymm_body,
        in_specs=[
            pl.BlockSpec((_NCP, 128), lambda: (0, 0)),
            pl.BlockSpec((_NCP, 128), lambda: (0, 0)),
            pl.BlockSpec((128, 64), lambda: (0, 0)),
        ],
        out_specs=pl.BlockSpec((_NCP, 128), lambda: (0, 0)),
        out_shape=jax.ShapeDtypeStruct((_NCP, 128), jnp.float32),
    )(t, a, w)


def _qkv_body(t_ref, q_ref, id_ref, wq_ref, wk_ref, wv_ref, w2_ref, bqkv_ref,
              oq_ref, ok_ref, ov_ref):
    m = jax.nn.relu(t_ref[...] + q_ref[...])
    n2 = 2.0 * jnp.sum(m * m, axis=1, keepdims=True)
    inv = 1.0 / jnp.maximum(jnp.sqrt(n2), 1e-12)
    idc = id_ref[...]
    for wref, oref, col in ((wq_ref, oq_ref, 0), (wk_ref, ok_ref, 1),
                            (wv_ref, ov_ref, 2)):
        z = jnp.dot(m, wref[...], preferred_element_type=jnp.float32)
        zid = jnp.dot(idc, w2_ref[pl.ds(2 * col, 2), :],
                      preferred_element_type=jnp.float32)
        oref[...] = z * inv + zid + bqkv_ref[col, :]


def _tc_qkv(t2, q1, idp, wqs, wks, wvs, w2s, bqkv):
    return pl.pallas_call(
        _qkv_body,
        in_specs=[
            pl.BlockSpec((_NCP, 128), lambda: (0, 0)),
            pl.BlockSpec((_NCP, 128), lambda: (0, 0)),
            pl.BlockSpec((_NCP, 2), lambda: (0, 0)),
            pl.BlockSpec((128, 64), lambda: (0, 0)),
            pl.BlockSpec((128, 64), lambda: (0, 0)),
            pl.BlockSpec((128, 64), lambda: (0, 0)),
            pl.BlockSpec((6, 64), lambda: (0, 0)),
            pl.BlockSpec((3, 64), lambda: (0, 0)),
        ],
        out_specs=[
            pl.BlockSpec((_NCP, 64), lambda: (0, 0)),
            pl.BlockSpec((_NCP, 64), lambda: (0, 0)),
            pl.BlockSpec((_NCP, 64), lambda: (0, 0)),
        ],
        out_shape=[jax.ShapeDtypeStruct((_NCP, 64), jnp.float32)] * 3,
    )(t2, q1, idp, wqs, wks, wvs, w2s, bqkv)


def _attn_body(q_ref, k_ref, v_ref, vl_ref, o_ref):
    b = pl.program_id(0)
    qb = q_ref[0]
    kb = k_ref[0]
    vb = v_ref[0]
    scores = lax.dot_general(
        qb, kb, (((1,), (1,)), ((), ())), preferred_element_type=jnp.float32
    ) * 0.125
    valid = lax.broadcasted_iota(jnp.int32, (L, L), 1) < vl_ref[b]
    scores = jnp.where(valid, scores, -1e9)
    mx = jnp.max(scores, axis=-1, keepdims=True)
    e = jnp.exp(scores - mx)
    p = e / jnp.sum(e, axis=-1, keepdims=True)
    o_ref[0] = jnp.dot(p, vb, preferred_element_type=jnp.float32)


def _tc_attn(q3, k3, v3, valid_len):
    return pl.pallas_call(
        _attn_body,
        grid=(B,),
        in_specs=[
            pl.BlockSpec((1, L, GW), lambda b: (b, 0, 0)),
            pl.BlockSpec((1, L, GW), lambda b: (b, 0, 0)),
            pl.BlockSpec((1, L, GW), lambda b: (b, 0, 0)),
            pl.BlockSpec(memory_space=pltpu.SMEM),
        ],
        out_specs=pl.BlockSpec((1, L, GW), lambda b: (b, 0, 0)),
        out_shape=jax.ShapeDtypeStruct((B, L, GW), jnp.float32),
    )(q3, k3, v3, valid_len)


def kernel(x, identifier, cluster, valid_len, W0, b0, W1, b1, W2, b2, Wq, bq, Wk, bk, Wv, bv):
    f32 = jnp.float32
    cluster_pad = jnp.pad(cluster, (0, _NP - N), constant_values=NC)
    z64 = jnp.zeros((64, 64), f32)
    W1a = jnp.concatenate([W1[:SW], z64], axis=0)
    W1b = jnp.concatenate([W1[SW:], z64], axis=0)
    W2a = jnp.concatenate([W2[:SW], z64], axis=0)
    W2b = jnp.concatenate([W2[SW:], z64], axis=0)
    wqs = jnp.concatenate([Wq[:SW] + Wq[SW:2 * SW], z64], axis=0)
    wks = jnp.concatenate([Wk[:SW] + Wk[SW:2 * SW], z64], axis=0)
    wvs = jnp.concatenate([Wv[:SW] + Wv[SW:2 * SW], z64], axis=0)
    w2s = jnp.concatenate([Wq[2 * SW:], Wk[2 * SW:], Wv[2 * SW:]], axis=0)
    bqkv = jnp.stack([bq, bk, bv], axis=0)
    idp = jnp.pad(identifier, ((0, _NCP - NC), (0, 0)))
    zero_ncp = jnp.zeros((_NCP, 128), f32)

    S = _sc_prep(cluster_pad)
    h0 = _tc_mm0(x, W0, b0)
    T0 = _sc_segmax(h0, S, 0.0)
    u1 = _tc_mm1(h0, W1a, b1)
    Q0 = _tc_tinymm(T0, zero_ncp, W1b)
    E1 = _sc_gather(Q0, cluster_pad)
    T1 = _sc_segmax(u1, S, _NEG)
    u2 = _tc_mm2(u1, E1, W2a, b2)
    Q1 = _tc_tinymm(T1, Q0, W2b)
    T2 = _sc_segmax(u2, S, _NEG)
    q, k, v = _tc_qkv(T2, Q1, idp, wqs, wks, wvs, w2s, bqkv)
    q3 = q[:NC].reshape(B, L, GW)
    k3 = k[:NC].reshape(B, L, GW)
    v3 = v[:NC].reshape(B, L, GW)
    return _tc_attn(q3, k3, v3, valid_len)


# speedup vs baseline: 4.2837x; 1.0083x over previous
import functools

import jax
import jax.numpy as jnp
from jax import lax
from jax.experimental import pallas as pl
from jax.experimental.pallas import tpu as pltpu
from jax.experimental.pallas import tpu_sc as plsc

N = 100000
B = 50
L = 50
NC = B * L
SW = 64
GW = 64

_NW = 32
_NP = 102400
_RPW = 3200
_GCH = 128
_CW = 80
_NCP = 2560
_SLEN = 2576
_CH = 512
_BLK = 2048
_GRID = 49

_mesh = plsc.VectorSubcoreMesh(core_axis_name="c", subcore_axis_name="s")
_NEG = -1.0e30


def _wid():
    return lax.axis_index("s") * 2 + lax.axis_index("c")


def _iota16():
    return lax.broadcasted_iota(jnp.int32, (16,), 0)


def _sget(ref, i):
    return ref[pl.ds(i, 16)][0]


def _sc_prep(cluster_pad):
    @functools.partial(
        pl.kernel,
        out_type=jax.ShapeDtypeStruct((_SLEN,), jnp.int32),
        mesh=_mesh,
        scratch_types=[
            pltpu.VMEM((96,), jnp.int32),
            pltpu.VMEM((96,), jnp.int32),
            pltpu.VMEM((96,), jnp.int32),
            pltpu.SemaphoreType.DMA,
        ],
    )
    def k(cl_hbm, s_hbm, idx_v, val_v, res_v, sem):
        w = _wid()
        cbase = pl.multiple_of(w * _CW, 16)
        it16 = _iota16()
        cs = [cbase + 16 * j + it16 for j in range(6)]
        pos = [jnp.zeros((16,), jnp.int32) for _ in range(6)]
        for st in range(16, -1, -1):
            step = jnp.int32(1 << st)
            cands = []
            for j in range(6):
                cand = jnp.minimum(pos[j] + step, jnp.int32(_NP))
                cands.append(cand)
                idx_v[pl.ds(16 * j, 16)] = cand - 1
            pltpu.async_copy(cl_hbm.at[idx_v], val_v, sem).wait()
            for j in range(6):
                pred = val_v[pl.ds(16 * j, 16)] < cs[j]
                pos[j] = jnp.where(pred, cands[j], pos[j])
        for j in range(6):
            res_v[pl.ds(16 * j, 16)] = pos[j]

        @pl.when(w == _NW - 1)
        def _():
            pltpu.sync_copy(res_v, s_hbm.at[pl.ds(cbase, 96)])

        @pl.when(w != _NW - 1)
        def _():
            pltpu.sync_copy(res_v.at[pl.ds(0, 80)], s_hbm.at[pl.ds(cbase, 80)])

    return k(cluster_pad)


def _sc_segmax(u, S, init):
    @functools.partial(
        pl.kernel,
        out_type=jax.ShapeDtypeStruct((_NCP, 128), jnp.float32),
        mesh=_mesh,
        scratch_types=[
            pltpu.VMEM((128,), jnp.int32),
            pltpu.VMEM((_CH, 128), jnp.float32),
            pltpu.VMEM((_CW, 128), jnp.float32),
            pltpu.SemaphoreType.DMA,
        ],
    )
    def k(u_hbm, s_hbm, t_hbm, sv, ubuf, tbuf, sem):
        w = _wid()
        c0 = pl.multiple_of(w * _CW, 16)
        pltpu.sync_copy(s_hbm.at[pl.ds(c0, 96)], sv.at[pl.ds(0, 96)])
        initv = jnp.full((16,), init, jnp.float32)
        zv = jnp.zeros((16,), jnp.float32)

        def prefill(i, carry):
            for t in range(4):
                tbuf[i, pl.ds(16 * t, 16)] = initv
            for t in range(4, 8):
                tbuf[i, pl.ds(16 * t, 16)] = zv
            return carry

        lax.fori_loop(0, _CW, prefill, 0)

        rs = jnp.bitwise_and(_sget(sv, jnp.int32(0)), jnp.int32(-8))
        re = _sget(sv, jnp.int32(_CW))
        nch = lax.div(re - rs + jnp.int32(_CH - 1), jnp.int32(_CH))

        def chunk(kk, cl):
            base = jnp.minimum(rs + kk * _CH, jnp.int32(_NP - _CH))
            base = pl.multiple_of(base, 8)
            pltpu.sync_copy(u_hbm.at[pl.ds(base, _CH)], ubuf)
            lim = base + _CH

            def redmem(cidx, s, e):
                def rbody(r, acc):
                    off = r - base
                    return tuple(
                        jnp.maximum(acc[t], ubuf[off, pl.ds(16 * t, 16)])
                        for t in range(4)
                    )

                acc0 = tuple(tbuf[cidx, pl.ds(16 * t, 16)] for t in range(4))
                acc = lax.fori_loop(s, e, rbody, acc0)
                for t in range(4):
                    tbuf[cidx, pl.ds(16 * t, 16)] = acc[t]

            def bit(_, lh):
                lo, hi = lh
                mid = lax.div(lo + hi, jnp.int32(2))
                gt = _sget(sv, mid + 1) > lim
                lo2 = jnp.where(gt, lo, mid + 1)
                hi2 = jnp.where(gt, mid, hi)
                keep = lo < hi
                return (jnp.where(keep, lo2, lo), jnp.where(keep, hi2, hi))

            cl_new, _ = lax.fori_loop(0, 7, bit, (cl, jnp.int32(_CW)))

            def cbody(c, z):
                s = jnp.maximum(_sget(sv, c), base)
                redmem(c, s, _sget(sv, c + 1))
                return z

            lax.fori_loop(cl, cl_new, cbody, 0)
            cl = cl_new
            cli = jnp.minimum(cl, jnp.int32(_CW - 1))
            s = jnp.maximum(_sget(sv, cli), base)
            e = jnp.minimum(_sget(sv, cli + 1), lim)
            e = jnp.where(cl < _CW, jnp.maximum(e, s), s)
            redmem(cli, s, e)
            return cl

        lax.fori_loop(0, nch, chunk, jnp.int32(0))
        pltpu.sync_copy(tbuf, t_hbm.at[pl.ds(c0, _CW)])

    return k(u, S)


def _sc_gather(table, idx):
    @functools.partial(
        pl.kernel,
        out_type=jax.ShapeDtypeStruct((_NP, 128), jnp.float32),
        mesh=_mesh,
        scratch_types=[
            pltpu.VMEM((_GCH,), jnp.int32),
            pltpu.VMEM((_GCH, 128), jnp.float32),
            pltpu.VMEM_SHARED((_NCP, 128), jnp.float32),
            pltpu.SemaphoreType.DMA,
        ],
    )
    def k(table_hbm, idx_hbm, out_hbm, idx_v, rows_v, qtab, sem):
        base = _wid() * _RPW

        @pl.when(lax.axis_index("s") == 0)
        def _():
            pltpu.sync_copy(table_hbm, qtab)

        plsc.subcore_barrier()

        def body(j, carry):
            off = base + j * _GCH
            pltpu.sync_copy(idx_hbm.at[pl.ds(off, _GCH)], idx_v)
            pltpu.async_copy(qtab.at[idx_v], rows_v, sem).wait()
            pltpu.sync_copy(rows_v, out_hbm.at[pl.ds(off, _GCH)])
            return carry

        lax.fori_loop(0, _RPW // _GCH, body, 0)

    return k(table, idx)


def _mm0_body(x_ref, w_ref, b_ref, o_ref):
    acc = jnp.dot(x_ref[...], w_ref[...], preferred_element_type=jnp.float32)
    h = jax.nn.relu(acc + b_ref[...])
    o_ref[...] = jnp.concatenate(
        [h, jnp.zeros((h.shape[0], 64), jnp.float32)], axis=1
    )


def _tc_mm0(x, w, b):
    return pl.pallas_call(
        _mm0_body,
        grid=(_GRID,),
        in_specs=[
            pl.BlockSpec((_BLK, 128), lambda i: (i, 0)),
            pl.BlockSpec((128, 64), lambda i: (0, 0)),
            pl.BlockSpec((64,), lambda i: (0,)),
        ],
        out_specs=pl.BlockSpec((_BLK, 128), lambda i: (i, 0)),
        out_shape=jax.ShapeDtypeStruct((_NP, 128), jnp.float32),
    )(x, w, b)


def _mm1_body(h_ref, w_ref, b_ref, o_ref):
    acc = jnp.dot(h_ref[...], w_ref[...], preferred_element_type=jnp.float32)
    u = acc + b_ref[...]
    o_ref[...] = jnp.concatenate(
        [u, jnp.zeros((u.shape[0], 64), jnp.float32)], axis=1
    )


def _tc_mm1(h, w, b):
    return pl.pallas_call(
        _mm1_body,
        grid=(_GRID,),
        in_specs=[
            pl.BlockSpec((_BLK, 128), lambda i: (i, 0)),
            pl.BlockSpec((128, 64), lambda i: (0, 0)),
            pl.BlockSpec((64,), lambda i: (0,)),
        ],
        out_specs=pl.BlockSpec((_BLK, 128), lambda i: (i, 0)),
        out_shape=jax.ShapeDtypeStruct((_NP, 128), jnp.float32),
    )(h, w, b)


def _mm2_body(u_ref, e_ref, w_ref, b_ref, o_ref):
    h = jax.nn.relu(u_ref[...] + e_ref[...])
    acc = jnp.dot(h, w_ref[...], preferred_element_type=jnp.float32)
    u = acc + b_ref[...]
    o_ref[...] = jnp.concatenate(
        [u, jnp.zeros((u.shape[0], 64), jnp.float32)], axis=1
    )


def _tc_mm2(u1, e1, w, b):
    return pl.pallas_call(
        _mm2_body,
        grid=(_GRID,),
        in_specs=[
            pl.BlockSpec((_BLK, 128), lambda i: (i, 0)),
            pl.BlockSpec((_BLK, 128), lambda i: (i, 0)),
            pl.BlockSpec((128, 64), lambda i: (0, 0)),
            pl.BlockSpec((64,), lambda i: (0,)),
        ],
        out_specs=pl.BlockSpec((_BLK, 128), lambda i: (i, 0)),
        out_shape=jax.ShapeDtypeStruct((_NP, 128), jnp.float32),
    )(u1, e1, w, b)


def _tinymm_body(t_ref, a_ref, w_ref, o_ref):
    m = jax.nn.relu(t_ref[...] + a_ref[...])
    acc = jnp.dot(m, w_ref[...], preferred_element_type=jnp.float32)
    o_ref[...] = jnp.concatenate(
        [acc, jnp.zeros((acc.shape[0], 64), jnp.float32)], axis=1
    )


def _tc_tinymm(t, a, w):
    return pl.pallas_call(
        _tinymm_body,
        in_specs=[
            pl.BlockSpec((_NCP, 128), lambda: (0, 0)),
            pl.BlockSpec((_NCP, 128), lambda: (0, 0)),
            pl.BlockSpec((128, 64), lambda: (0, 0)),
        ],
        out_specs=pl.BlockSpec((_NCP, 128), lambda: (0, 0)),
        out_shape=jax.ShapeDtypeStruct((_NCP, 128), jnp.float32),
    )(t, a, w)


def _qkv_body(t_ref, q_ref, id_ref, wq_ref, wk_ref, wv_ref, w2_ref, bqkv_ref,
              oq_ref, ok_ref, ov_ref):
    m = jax.nn.relu(t_ref[...] + q_ref[...])
    n2 = 2.0 * jnp.sum(m * m, axis=1, keepdims=True)
    inv = 1.0 / jnp.maximum(jnp.sqrt(n2), 1e-12)
    idc = id_ref[...]
    for wref, oref, col in ((wq_ref, oq_ref, 0), (wk_ref, ok_ref, 1),
                            (wv_ref, ov_ref, 2)):
        z = jnp.dot(m, wref[...], preferred_element_type=jnp.float32)
        zid = jnp.dot(idc, w2_ref[pl.ds(2 * col, 2), :],
                      preferred_element_type=jnp.float32)
        oref[...] = z * inv + zid + bqkv_ref[col, :]


def _tc_qkv(t2, q1, idp, wqs, wks, wvs, w2s, bqkv):
    return pl.pallas_call(
        _qkv_body,
        in_specs=[
            pl.BlockSpec((_NCP, 128), lambda: (0, 0)),
            pl.BlockSpec((_NCP, 128), lambda: (0, 0)),
            pl.BlockSpec((_NCP, 2), lambda: (0, 0)),
            pl.BlockSpec((128, 64), lambda: (0, 0)),
            pl.BlockSpec((128, 64), lambda: (0, 0)),
            pl.BlockSpec((128, 64), lambda: (0, 0)),
            pl.BlockSpec((6, 64), lambda: (0, 0)),
            pl.BlockSpec((3, 64), lambda: (0, 0)),
        ],
        out_specs=[
            pl.BlockSpec((_NCP, 64), lambda: (0, 0)),
            pl.BlockSpec((_NCP, 64), lambda: (0, 0)),
            pl.BlockSpec((_NCP, 64), lambda: (0, 0)),
        ],
        out_shape=[jax.ShapeDtypeStruct((_NCP, 64), jnp.float32)] * 3,
    )(t2, q1, idp, wqs, wks, wvs, w2s, bqkv)


def _attn_body(q_ref, k_ref, v_ref, vl_ref, o_ref):
    b = pl.program_id(0)
    qb = q_ref[0]
    kb = k_ref[0]
    vb = v_ref[0]
    scores = lax.dot_general(
        qb, kb, (((1,), (1,)), ((), ())), preferred_element_type=jnp.float32
    ) * 0.125
    valid = lax.broadcasted_iota(jnp.int32, (L, L), 1) < vl_ref[b]
    scores = jnp.where(valid, scores, -1e9)
    mx = jnp.max(scores, axis=-1, keepdims=True)
    e = jnp.exp(scores - mx)
    p = e / jnp.sum(e, axis=-1, keepdims=True)
    o_ref[0] = jnp.dot(p, vb, preferred_element_type=jnp.float32)


def _tc_attn(q3, k3, v3, valid_len):
    return pl.pallas_call(
        _attn_body,
        grid=(B,),
        in_specs=[
            pl.BlockSpec((1, L, GW), lambda b: (b, 0, 0)),
            pl.BlockSpec((1, L, GW), lambda b: (b, 0, 0)),
            pl.BlockSpec((1, L, GW), lambda b: (b, 0, 0)),
            pl.BlockSpec(memory_space=pltpu.SMEM),
        ],
        out_specs=pl.BlockSpec((1, L, GW), lambda b: (b, 0, 0)),
        out_shape=jax.ShapeDtypeStruct((B, L, GW), jnp.float32),
    )(q3, k3, v3, valid_len)


def kernel(x, identifier, cluster, valid_len, W0, b0, W1, b1, W2, b2, Wq, bq, Wk, bk, Wv, bv):
    f32 = jnp.float32
    cluster_pad = jnp.pad(cluster, (0, _NP - N), constant_values=NC)
    z64 = jnp.zeros((64, 64), f32)
    W1a = jnp.concatenate([W1[:SW], z64], axis=0)
    W1b = jnp.concatenate([W1[SW:], z64], axis=0)
    W2a = jnp.concatenate([W2[:SW], z64], axis=0)
    W2b = jnp.concatenate([W2[SW:], z64], axis=0)
    wqs = jnp.concatenate([Wq[:SW] + Wq[SW:2 * SW], z64], axis=0)
    wks = jnp.concatenate([Wk[:SW] + Wk[SW:2 * SW], z64], axis=0)
    wvs = jnp.concatenate([Wv[:SW] + Wv[SW:2 * SW], z64], axis=0)
    w2s = jnp.concatenate([Wq[2 * SW:], Wk[2 * SW:], Wv[2 * SW:]], axis=0)
    bqkv = jnp.stack([bq, bk, bv], axis=0)
    idp = jnp.pad(identifier, ((0, _NCP - NC), (0, 0)))
    zero_ncp = jnp.zeros((_NCP, 128), f32)

    S = _sc_prep(cluster_pad)
    h0 = _tc_mm0(x, W0, b0)
    T0 = _sc_segmax(h0, S, 0.0)
    u1 = _tc_mm1(h0, W1a, b1)
    Q0 = _tc_tinymm(T0, zero_ncp, W1b)
    E1 = _sc_gather(Q0, cluster_pad)
    T1 = _sc_segmax(u1, S, _NEG)
    u2 = _tc_mm2(u1, E1, W2a, b2)
    Q1 = _tc_tinymm(T1, Q0, W2b)
    T2 = _sc_segmax(u2, S, _NEG)
    q, k, v = _tc_qkv(T2, Q1, idp, wqs, wks, wvs, w2s, bqkv)
    q3 = q[:NC].reshape(B, L, GW)
    k3 = k[:NC].reshape(B, L, GW)
    v3 = v[:NC].reshape(B, L, GW)
    return _tc_attn(q3, k3, v3, valid_len)


# speedup vs baseline: 4.2901x; 1.0015x over previous
import functools

import jax
import jax.numpy as jnp
from jax import lax
from jax.experimental import pallas as pl
from jax.experimental.pallas import tpu as pltpu
from jax.experimental.pallas import tpu_sc as plsc

N = 100000
B = 50
L = 50
NC = B * L
SW = 64
GW = 64

_NW = 32
_NP = 102400
_RPW = 3200
_GCH = 128
_CW = 80
_NCP = 2560
_SLEN = 2576
_CH = 512
_BLK = 2048
_GRID = 49

_mesh = plsc.VectorSubcoreMesh(core_axis_name="c", subcore_axis_name="s")
_NEG = -1.0e30


def _wid():
    return lax.axis_index("s") * 2 + lax.axis_index("c")


def _iota16():
    return lax.broadcasted_iota(jnp.int32, (16,), 0)


def _sget(ref, i):
    return ref[pl.ds(i, 16)][0]


def _sc_prep(cluster_pad):
    @functools.partial(
        pl.kernel,
        out_type=jax.ShapeDtypeStruct((_SLEN,), jnp.int32),
        mesh=_mesh,
        scratch_types=[
            pltpu.VMEM((96,), jnp.int32),
            pltpu.VMEM((96,), jnp.int32),
            pltpu.VMEM((96,), jnp.int32),
            pltpu.SemaphoreType.DMA,
        ],
    )
    def k(cl_hbm, s_hbm, idx_v, val_v, res_v, sem):
        w = _wid()
        cbase = pl.multiple_of(w * _CW, 16)
        it16 = _iota16()
        cs = [cbase + 16 * j + it16 for j in range(6)]
        pos = [jnp.zeros((16,), jnp.int32) for _ in range(6)]
        for st in range(16, -1, -1):
            step = jnp.int32(1 << st)
            cands = []
            for j in range(6):
                cand = jnp.minimum(pos[j] + step, jnp.int32(_NP))
                cands.append(cand)
                idx_v[pl.ds(16 * j, 16)] = cand - 1
            pltpu.async_copy(cl_hbm.at[idx_v], val_v, sem).wait()
            for j in range(6):
                pred = val_v[pl.ds(16 * j, 16)] < cs[j]
                pos[j] = jnp.where(pred, cands[j], pos[j])
        for j in range(6):
            res_v[pl.ds(16 * j, 16)] = pos[j]

        @pl.when(w == _NW - 1)
        def _():
            pltpu.sync_copy(res_v, s_hbm.at[pl.ds(cbase, 96)])

        @pl.when(w != _NW - 1)
        def _():
            pltpu.sync_copy(res_v.at[pl.ds(0, 80)], s_hbm.at[pl.ds(cbase, 80)])

    return k(cluster_pad)


def _sc_segmax(u, S, init):
    @functools.partial(
        pl.kernel,
        out_type=jax.ShapeDtypeStruct((_NCP, 128), jnp.float32),
        mesh=_mesh,
        scratch_types=[
            pltpu.VMEM((128,), jnp.int32),
            pltpu.VMEM((_CH, 128), jnp.float32),
            pltpu.VMEM((_CW, 128), jnp.float32),
            pltpu.SemaphoreType.DMA,
        ],
    )
    def k(u_hbm, s_hbm, t_hbm, sv, ubuf, tbuf, sem):
        w = _wid()
        c0 = pl.multiple_of(w * _CW, 16)
        pltpu.sync_copy(s_hbm.at[pl.ds(c0, 96)], sv.at[pl.ds(0, 96)])
        initv = jnp.full((16,), init, jnp.float32)
        zv = jnp.zeros((16,), jnp.float32)

        def prefill(i, carry):
            for t in range(4):
                tbuf[i, pl.ds(16 * t, 16)] = initv
            for t in range(4, 8):
                tbuf[i, pl.ds(16 * t, 16)] = zv
            return carry

        lax.fori_loop(0, _CW, prefill, 0)

        rs = jnp.bitwise_and(_sget(sv, jnp.int32(0)), jnp.int32(-8))
        re = _sget(sv, jnp.int32(_CW))
        nch = lax.div(re - rs + jnp.int32(_CH - 1), jnp.int32(_CH))

        def chunk(kk, cl):
            base = jnp.minimum(rs + kk * _CH, jnp.int32(_NP - _CH))
            base = pl.multiple_of(base, 8)
            pltpu.sync_copy(u_hbm.at[pl.ds(base, _CH)], ubuf)
            lim = base + _CH

            def redmem(cidx, s, e):
                def rbody(r, acc):
                    off = r - base
                    return tuple(
                        jnp.maximum(acc[t], ubuf[off, pl.ds(16 * t, 16)])
                        for t in range(4)
                    )

                acc0 = tuple(tbuf[cidx, pl.ds(16 * t, 16)] for t in range(4))
                acc = lax.fori_loop(s, e, rbody, acc0)
                for t in range(4):
                    tbuf[cidx, pl.ds(16 * t, 16)] = acc[t]

            def bit(_, lh):
                lo, hi = lh
                mid = lax.div(lo + hi, jnp.int32(2))
                gt = _sget(sv, mid + 1) > lim
                lo2 = jnp.where(gt, lo, mid + 1)
                hi2 = jnp.where(gt, mid, hi)
                keep = lo < hi
                return (jnp.where(keep, lo2, lo), jnp.where(keep, hi2, hi))

            cl_new, _ = lax.fori_loop(0, 7, bit, (cl, jnp.int32(_CW)))

            def cbody(c, z):
                s = jnp.maximum(_sget(sv, c), base)
                redmem(c, s, _sget(sv, c + 1))
                return z

            lax.fori_loop(cl, cl_new, cbody, 0)
            cl = cl_new
            cli = jnp.minimum(cl, jnp.int32(_CW - 1))
            s = jnp.maximum(_sget(sv, cli), base)
            e = jnp.minimum(_sget(sv, cli + 1), lim)
            e = jnp.where(cl < _CW, jnp.maximum(e, s), s)
            redmem(cli, s, e)
            return cl

        lax.fori_loop(0, nch, chunk, jnp.int32(0))
        pltpu.sync_copy(tbuf, t_hbm.at[pl.ds(c0, _CW)])

    return k(u, S)


def _sc_gather(table, idx):
    @functools.partial(
        pl.kernel,
        out_type=jax.ShapeDtypeStruct((_NP, 128), jnp.float32),
        mesh=_mesh,
        scratch_types=[
            pltpu.VMEM((_RPW,), jnp.int32),
            pltpu.VMEM((_GCH, 128), jnp.float32),
            pltpu.VMEM((_GCH, 128), jnp.float32),
            pltpu.VMEM_SHARED((_NCP, 128), jnp.float32),
            pltpu.SemaphoreType.DMA,
            pltpu.SemaphoreType.DMA,
        ],
    )
    def k(table_hbm, idx_hbm, out_hbm, idx_all, rows0, rows1, qtab, semg, semo):
        base = pl.multiple_of(_wid() * _RPW, 128)
        pltpu.sync_copy(idx_hbm.at[pl.ds(base, _RPW)], idx_all)

        @pl.when(lax.axis_index("s") == 0)
        def _():
            pltpu.sync_copy(table_hbm, qtab)

        plsc.subcore_barrier()
        nj = _RPW // _GCH

        def step(j, rows_v):
            off = base + j * _GCH

            @pl.when(j >= 2)
            def _():
                pltpu.make_async_copy(
                    rows_v, out_hbm.at[pl.ds(off, _GCH)], semo
                ).wait()

            pltpu.async_copy(
                qtab.at[idx_all.at[pl.ds(j * _GCH, _GCH)]], rows_v, semg
            ).wait()
            pltpu.async_copy(rows_v, out_hbm.at[pl.ds(off, _GCH)], semo)

        def body(j, carry):
            @pl.when(lax.rem(j, jnp.int32(2)) == 0)
            def _():
                step(j, rows0)

            @pl.when(lax.rem(j, jnp.int32(2)) == 1)
            def _():
                step(j, rows1)

            return carry

        lax.fori_loop(0, nj, body, 0)
        pltpu.make_async_copy(rows0, out_hbm.at[pl.ds(base, _GCH)], semo).wait()
        pltpu.make_async_copy(rows1, out_hbm.at[pl.ds(base, _GCH)], semo).wait()

    return k(table, idx)


def _mm0_body(x_ref, w_ref, b_ref, o_ref):
    acc = jnp.dot(x_ref[...], w_ref[...], preferred_element_type=jnp.float32)
    h = jax.nn.relu(acc + b_ref[...])
    o_ref[...] = jnp.concatenate(
        [h, jnp.zeros((h.shape[0], 64), jnp.float32)], axis=1
    )


def _tc_mm0(x, w, b):
    return pl.pallas_call(
        _mm0_body,
        grid=(_GRID,),
        in_specs=[
            pl.BlockSpec((_BLK, 128), lambda i: (i, 0)),
            pl.BlockSpec((128, 64), lambda i: (0, 0)),
            pl.BlockSpec((64,), lambda i: (0,)),
        ],
        out_specs=pl.BlockSpec((_BLK, 128), lambda i: (i, 0)),
        out_shape=jax.ShapeDtypeStruct((_NP, 128), jnp.float32),
    )(x, w, b)


def _mm1_body(h_ref, w_ref, b_ref, o_ref):
    acc = jnp.dot(h_ref[...], w_ref[...], preferred_element_type=jnp.float32)
    u = acc + b_ref[...]
    o_ref[...] = jnp.concatenate(
        [u, jnp.zeros((u.shape[0], 64), jnp.float32)], axis=1
    )


def _tc_mm1(h, w, b):
    return pl.pallas_call(
        _mm1_body,
        grid=(_GRID,),
        in_specs=[
            pl.BlockSpec((_BLK, 128), lambda i: (i, 0)),
            pl.BlockSpec((128, 64), lambda i: (0, 0)),
            pl.BlockSpec((64,), lambda i: (0,)),
        ],
        out_specs=pl.BlockSpec((_BLK, 128), lambda i: (i, 0)),
        out_shape=jax.ShapeDtypeStruct((_NP, 128), jnp.float32),
    )(h, w, b)


def _mm2_body(u_ref, e_ref, w_ref, b_ref, o_ref):
    h = jax.nn.relu(u_ref[...] + e_ref[...])
    acc = jnp.dot(h, w_ref[...], preferred_element_type=jnp.float32)
    u = acc + b_ref[...]
    o_ref[...] = jnp.concatenate(
        [u, jnp.zeros((u.shape[0], 64), jnp.float32)], axis=1
    )


def _tc_mm2(u1, e1, w, b):
    return pl.pallas_call(
        _mm2_body,
        grid=(_GRID,),
        in_specs=[
            pl.BlockSpec((_BLK, 128), lambda i: (i, 0)),
            pl.BlockSpec((_BLK, 128), lambda i: (i, 0)),
            pl.BlockSpec((128, 64), lambda i: (0, 0)),
            pl.BlockSpec((64,), lambda i: (0,)),
        ],
        out_specs=pl.BlockSpec((_BLK, 128), lambda i: (i, 0)),
        out_shape=jax.ShapeDtypeStruct((_NP, 128), jnp.float32),
    )(u1, e1, w, b)


def _tinymm_body(t_ref, a_ref, w_ref, o_ref):
    m = jax.nn.relu(t_ref[...] + a_ref[...])
    acc = jnp.dot(m, w_ref[...], preferred_element_type=jnp.float32)
    o_ref[...] = jnp.concatenate(
        [acc, jnp.zeros((acc.shape[0], 64), jnp.float32)], axis=1
    )


def _tc_tinymm(t, a, w):
    return pl.pallas_call(
        _tinymm_body,
        in_specs=[
            pl.BlockSpec((_NCP, 128), lambda: (0, 0)),
            pl.BlockSpec((_NCP, 128), lambda: (0, 0)),
            pl.BlockSpec((128, 64), lambda: (0, 0)),
        ],
        out_specs=pl.BlockSpec((_NCP, 128), lambda: (0, 0)),
        out_shape=jax.ShapeDtypeStruct((_NCP, 128), jnp.float32),
    )(t, a, w)


def _qkv_body(t_ref, q_ref, id_ref, wq_ref, wk_ref, wv_ref, w2_ref, bqkv_ref,
              oq_ref, ok_ref, ov_ref):
    m = jax.nn.relu(t_ref[...] + q_ref[...])
    n2 = 2.0 * jnp.sum(m * m, axis=1, keepdims=True)
    inv = 1.0 / jnp.maximum(jnp.sqrt(n2), 1e-12)
    idc = id_ref[...]
    for wref, oref, col in ((wq_ref, oq_ref, 0), (wk_ref, ok_ref, 1),
                            (wv_ref, ov_ref, 2)):
        z = jnp.dot(m, wref[...], preferred_element_type=jnp.float32)
        zid = jnp.dot(idc, w2_ref[pl.ds(2 * col, 2), :],
                      preferred_element_type=jnp.float32)
        oref[...] = z * inv + zid + bqkv_ref[col, :]


def _tc_qkv(t2, q1, idp, wqs, wks, wvs, w2s, bqkv):
    return pl.pallas_call(
        _qkv_body,
        in_specs=[
            pl.BlockSpec((_NCP, 128), lambda: (0, 0)),
            pl.BlockSpec((_NCP, 128), lambda: (0, 0)),
            pl.BlockSpec((_NCP, 2), lambda: (0, 0)),
            pl.BlockSpec((128, 64), lambda: (0, 0)),
            pl.BlockSpec((128, 64), lambda: (0, 0)),
            pl.BlockSpec((128, 64), lambda: (0, 0)),
            pl.BlockSpec((6, 64), lambda: (0, 0)),
            pl.BlockSpec((3, 64), lambda: (0, 0)),
        ],
        out_specs=[
            pl.BlockSpec((_NCP, 64), lambda: (0, 0)),
            pl.BlockSpec((_NCP, 64), lambda: (0, 0)),
            pl.BlockSpec((_NCP, 64), lambda: (0, 0)),
        ],
        out_shape=[jax.ShapeDtypeStruct((_NCP, 64), jnp.float32)] * 3,
    )(t2, q1, idp, wqs, wks, wvs, w2s, bqkv)


def _attn_body(q_ref, k_ref, v_ref, vl_ref, o_ref):
    b = pl.program_id(0)
    qb = q_ref[0]
    kb = k_ref[0]
    vb = v_ref[0]
    scores = lax.dot_general(
        qb, kb, (((1,), (1,)), ((), ())), preferred_element_type=jnp.float32
    ) * 0.125
    valid = lax.broadcasted_iota(jnp.int32, (L, L), 1) < vl_ref[b]
    scores = jnp.where(valid, scores, -1e9)
    mx = jnp.max(scores, axis=-1, keepdims=True)
    e = jnp.exp(scores - mx)
    p = e / jnp.sum(e, axis=-1, keepdims=True)
    o_ref[0] = jnp.dot(p, vb, preferred_element_type=jnp.float32)


def _tc_attn(q3, k3, v3, valid_len):
    return pl.pallas_call(
        _attn_body,
        grid=(B,),
        in_specs=[
            pl.BlockSpec((1, L, GW), lambda b: (b, 0, 0)),
            pl.BlockSpec((1, L, GW), lambda b: (b, 0, 0)),
            pl.BlockSpec((1, L, GW), lambda b: (b, 0, 0)),
            pl.BlockSpec(memory_space=pltpu.SMEM),
        ],
        out_specs=pl.BlockSpec((1, L, GW), lambda b: (b, 0, 0)),
        out_shape=jax.ShapeDtypeStruct((B, L, GW), jnp.float32),
    )(q3, k3, v3, valid_len)


def kernel(x, identifier, cluster, valid_len, W0, b0, W1, b1, W2, b2, Wq, bq, Wk, bk, Wv, bv):
    f32 = jnp.float32
    cluster_pad = jnp.pad(cluster, (0, _NP - N), constant_values=NC)
    z64 = jnp.zeros((64, 64), f32)
    W1a = jnp.concatenate([W1[:SW], z64], axis=0)
    W1b = jnp.concatenate([W1[SW:], z64], axis=0)
    W2a = jnp.concatenate([W2[:SW], z64], axis=0)
    W2b = jnp.concatenate([W2[SW:], z64], axis=0)
    wqs = jnp.concatenate([Wq[:SW] + Wq[SW:2 * SW], z64], axis=0)
    wks = jnp.concatenate([Wk[:SW] + Wk[SW:2 * SW], z64], axis=0)
    wvs = jnp.concatenate([Wv[:SW] + Wv[SW:2 * SW], z64], axis=0)
    w2s = jnp.concatenate([Wq[2 * SW:], Wk[2 * SW:], Wv[2 * SW:]], axis=0)
    bqkv = jnp.stack([bq, bk, bv], axis=0)
    idp = jnp.pad(identifier, ((0, _NCP - NC), (0, 0)))
    zero_ncp = jnp.zeros((_NCP, 128), f32)

    S = _sc_prep(cluster_pad)
    h0 = _tc_mm0(x, W0, b0)
    T0 = _sc_segmax(h0, S, 0.0)
    u1 = _tc_mm1(h0, W1a, b1)
    Q0 = _tc_tinymm(T0, zero_ncp, W1b)
    E1 = _sc_gather(Q0, cluster_pad)
    T1 = _sc_segmax(u1, S, _NEG)
    u2 = _tc_mm2(u1, E1, W2a, b2)
    Q1 = _tc_tinymm(T1, Q0, W2b)
    T2 = _sc_segmax(u2, S, _NEG)
    q, k, v = _tc_qkv(T2, Q1, idp, wqs, wks, wvs, w2s, bqkv)
    q3 = q[:NC].reshape(B, L, GW)
    k3 = k[:NC].reshape(B, L, GW)
    v3 = v[:NC].reshape(B, L, GW)
    return _tc_attn(q3, k3, v3, valid_len)
